# baseline (speedup 1.0000x reference)
import functools

import jax
import jax.numpy as jnp
from jax import lax
from jax.experimental import pallas as pl
from jax.experimental.pallas import tpu as pltpu
from jax.experimental.pallas import tpu_sc as plsc

N = 10000
E = 160000
DN = 128
H = 16
K = 16

NC = 2
NS = 16
NW = NC * NS
CH = 128
NCHUNK = 40
EPW = NCHUNK * CH
ET = NW * EPW
NBUF = 2
NA = 10240
RPT = NA // NS
ZB = 64
GW = K * H


def _make_sc_kernel():
    mesh = plsc.VectorSubcoreMesh(core_axis_name="c", subcore_axis_name="s")

    @functools.partial(
        pl.kernel,
        out_type=jax.ShapeDtypeStruct((NC, NS, RPT, H), jnp.float32),
        mesh=mesh,
        scratch_types=[
            pltpu.VMEM((NCHUNK, CH), jnp.int32),
            pltpu.VMEM((NCHUNK, CH), jnp.int32),
            pltpu.VMEM((K, K), jnp.float32),
            pltpu.VMEM((1, K), jnp.float32),
            pltpu.VMEM((NBUF, CH, K), jnp.float32),
            pltpu.VMEM((NBUF, CH, GW), jnp.float32),
            pltpu.VMEM((NBUF, CH, H), jnp.float32),
            pltpu.VMEM((NBUF, CH, H), jnp.float32),
            pltpu.VMEM((ZB, H), jnp.float32),
            pltpu.VMEM_SHARED((NA, H), jnp.float32),
            pltpu.SemaphoreType.DMA,
            pltpu.SemaphoreType.DMA,
            pltpu.SemaphoreType.DMA,
            pltpu.SemaphoreType.DMA,
            pltpu.SemaphoreType.DMA,
            pltpu.SemaphoreType.DMA,
            pltpu.SemaphoreType.DMA,
            pltpu.SemaphoreType.DMA,
        ],
        compiler_params=pltpu.CompilerParams(use_tc_tiling_on_sc=False),
    )
    def sc_kernel(g_hbm, hb_hbm, ea_hbm, w_hbm, b_hbm, src_hbm, dst_hbm,
                  parts_hbm,
                  src_v, dst_v, w_v, b_v, ea_v, g_v, hb_v, msg_v, zero_v,
                  agg_sh, gsem0, gsem1, bsem0, bsem1, esem0, esem1,
                  ssem0, ssem1):
        cid = lax.axis_index("c")
        sid = lax.axis_index("s")
        wid = sid * NC + cid
        gsems = (gsem0, gsem1)
        bsems = (bsem0, bsem1)
        esems = (esem0, esem1)
        ssems = (ssem0, ssem1)

        pltpu.sync_copy(src_hbm.at[wid], src_v)
        pltpu.sync_copy(dst_hbm.at[wid], dst_v)
        pltpu.sync_copy(w_hbm, w_v)
        pltpu.sync_copy(b_hbm, b_v)
        wrows = [w_v[i, :] for i in range(K)]
        bvec = b_v[0, :]

        def _zfill(j, c):
            zero_v[j, :] = jnp.zeros((H,), jnp.float32)
            return c
        lax.fori_loop(0, ZB, _zfill, 0)

        def _zcopy(j, c):
            pltpu.sync_copy(zero_v, agg_sh.at[pl.ds(sid * RPT + j * ZB, ZB)])
            return c
        lax.fori_loop(0, RPT // ZB, _zcopy, 0)
        plsc.subcore_barrier()

        def _start(t, b):
            pltpu.async_copy(g_hbm.at[src_v.at[t]], g_v.at[b], gsems[b])
            pltpu.async_copy(hb_hbm.at[src_v.at[t]], hb_v.at[b], bsems[b])
            pltpu.async_copy(ea_hbm.at[wid, t], ea_v.at[b], esems[b])

        def _wait(t, b):
            pltpu.make_async_copy(g_hbm.at[src_v.at[t]], g_v.at[b],
                                  gsems[b]).wait()
            pltpu.make_async_copy(hb_hbm.at[src_v.at[t]], hb_v.at[b],
                                  bsems[b]).wait()
            pltpu.make_async_copy(ea_hbm.at[wid, t], ea_v.at[b],
                                  esems[b]).wait()

        def _scatter_wait(t, b):
            pltpu.make_async_copy(msg_v.at[b], agg_sh.at[dst_v.at[t]],
                                  ssems[b]).wait()

        def _compute(b):
            def _edge(e, c):
                earow = ea_v[b, e, :]
                hacc = [bvec + earow[0] * wrows[0], earow[1] * wrows[1],
                        earow[2] * wrows[2], earow[3] * wrows[3]]
                for i in range(4, K):
                    hacc[i % 4] = hacc[i % 4] + earow[i] * wrows[i]
                hrow = jnp.maximum((hacc[0] + hacc[1]) + (hacc[2] + hacc[3]),
                                   0.0)

                def term(k):
                    return hrow[k] * g_v[b, e, pl.ds(k * H, H)]
                acc = [hb_v[b, e, :] + term(0), term(1), term(2), term(3)]
                for k in range(4, K):
                    acc[k % 4] = acc[k % 4] + term(k)
                msg_v[b, e, :] = (acc[0] + acc[1]) + (acc[2] + acc[3])
                return c
            lax.fori_loop(0, CH, _edge, 0)

        for b in range(NBUF):
            _start(b, b)

        def _group(gi, c):
            for b in range(NBUF):
                t = gi * NBUF + b
                _wait(t, b)
                _compute(b)
                pltpu.sync_copy(msg_v.at[b], agg_sh.at[dst_v.at[t]], add=True)

                @pl.when(t + NBUF < NCHUNK)
                def _():
                    _start(t + NBUF, b)
            return c
        lax.fori_loop(0, NCHUNK // NBUF, _group, 0)

        plsc.subcore_barrier()
        pltpu.sync_copy(agg_sh.at[pl.ds(sid * RPT, RPT)],
                        parts_hbm.at[cid, sid])

    return sc_kernel


_sc_layer = _make_sc_kernel()


_BN = 2000


def _node_pre_body(x_ref, wt_ref, bt_ref, lin_ref, bias_ref,
                   g_ref, hb_ref, root_ref):
    xb = x_ref[...]
    g_ref[...] = jnp.dot(xb, wt_ref[...], preferred_element_type=jnp.float32)
    hb_ref[...] = jnp.dot(xb, bt_ref[...], preferred_element_type=jnp.float32)
    root_ref[...] = (
        jnp.dot(xb, lin_ref[...], preferred_element_type=jnp.float32)
        + bias_ref[...])


def _combine_pre_body(a0_ref, a1_ref, root_ref, wt_ref, bt_ref, lin_ref,
                      bias_ref, g_ref, hb_ref, root2_ref):
    hn = jnp.maximum(a0_ref[...] + a1_ref[...] + root_ref[...], 0.0)
    g_ref[...] = jnp.dot(hn, wt_ref[...], preferred_element_type=jnp.float32)
    hb_ref[...] = jnp.dot(hn, bt_ref[...], preferred_element_type=jnp.float32)
    root2_ref[...] = (
        jnp.dot(hn, lin_ref[...], preferred_element_type=jnp.float32)
        + bias_ref[...])


def _final_body(a0_ref, a1_ref, root_ref, cw_ref, cb_ref, out_ref):
    hn = jnp.maximum(a0_ref[...] + a1_ref[...] + root_ref[...], 0.0)
    logits = (jnp.dot(hn, cw_ref[...], preferred_element_type=jnp.float32)
              + cb_ref[...])
    m = jnp.max(logits, axis=1, keepdims=True)
    z = logits - m
    lse = jnp.log(jnp.sum(jnp.exp(z), axis=1, keepdims=True))
    out_ref[...] = z - lse


def _full(shape):
    return pl.BlockSpec(shape, lambda i: (0,) * len(shape))


def _node_pre(x, wt, bt, lin, bias):
    d = x.shape[1]
    grid = (N // _BN,)
    return pl.pallas_call(
        _node_pre_body,
        grid=grid,
        in_specs=[
            pl.BlockSpec((_BN, d), lambda i: (i, 0)),
            _full((d, GW)), _full((d, H)), _full((d, H)), _full((1, H)),
        ],
        out_specs=[pl.BlockSpec((_BN, GW), lambda i: (i, 0)),
                   pl.BlockSpec((_BN, H), lambda i: (i, 0)),
                   pl.BlockSpec((_BN, H), lambda i: (i, 0))],
        out_shape=[jax.ShapeDtypeStruct((N, GW), jnp.float32),
                   jax.ShapeDtypeStruct((N, H), jnp.float32),
                   jax.ShapeDtypeStruct((N, H), jnp.float32)],
    )(x, wt, bt, lin, bias)


def _combine_pre(a0, a1, root, wt, bt, lin, bias):
    grid = (N // _BN,)
    return pl.pallas_call(
        _combine_pre_body,
        grid=grid,
        in_specs=[
            pl.BlockSpec((_BN, H), lambda i: (i, 0)),
            pl.BlockSpec((_BN, H), lambda i: (i, 0)),
            pl.BlockSpec((_BN, H), lambda i: (i, 0)),
            _full((H, GW)), _full((H, H)), _full((H, H)), _full((1, H)),
        ],
        out_specs=[pl.BlockSpec((_BN, GW), lambda i: (i, 0)),
                   pl.BlockSpec((_BN, H), lambda i: (i, 0)),
                   pl.BlockSpec((_BN, H), lambda i: (i, 0))],
        out_shape=[jax.ShapeDtypeStruct((N, GW), jnp.float32),
                   jax.ShapeDtypeStruct((N, H), jnp.float32),
                   jax.ShapeDtypeStruct((N, H), jnp.float32)],
    )(a0, a1, root, wt, bt, lin, bias)


def _final(a0, a1, root, cw, cb):
    grid = (N // _BN,)
    return pl.pallas_call(
        _final_body,
        grid=grid,
        in_specs=[
            pl.BlockSpec((_BN, H), lambda i: (i, 0)),
            pl.BlockSpec((_BN, H), lambda i: (i, 0)),
            pl.BlockSpec((_BN, H), lambda i: (i, 0)),
            _full((H, 2)), _full((1, 2)),
        ],
        out_specs=pl.BlockSpec((_BN, 2), lambda i: (i, 0)),
        out_shape=jax.ShapeDtypeStruct((N, 2), jnp.float32),
    )(a0, a1, root, cw, cb)


def kernel(x, edge_index, edge_attr, nn1_w1, nn1_b1, nn1_w2, nn1_b2, lin1,
           bias1, nn2_w1, nn2_b1, nn2_w2, nn2_b2, lin2, bias2, cls_w, cls_b):
    wt1 = nn1_w2.reshape(K, DN, H).transpose(1, 0, 2).reshape(DN, K * H)
    bt1 = nn1_b2.reshape(DN, H)
    wt2 = nn2_w2.reshape(K, H, H).transpose(1, 0, 2).reshape(H, K * H)
    bt2 = nn2_b2.reshape(H, H)

    pad = ET - E
    src = jnp.concatenate(
        [edge_index[0], jnp.zeros((pad,), jnp.int32)]).reshape(NW, NCHUNK, CH)
    dst = jnp.concatenate(
        [edge_index[1], jnp.full((pad,), N, jnp.int32)]).reshape(NW, NCHUNK,
                                                                 CH)
    ea = jnp.concatenate(
        [edge_attr, jnp.zeros((pad, K), jnp.float32)]).reshape(NW, NCHUNK,
                                                               CH, K)

    g1, hb1, root1 = _node_pre(x, wt1, bt1, lin1, bias1.reshape(1, H))
    parts1 = _sc_layer(g1, hb1, ea, nn1_w1, nn1_b1.reshape(1, K),
                       src, dst).reshape(NC, NA, H)

    g2, hb2, root2 = _combine_pre(parts1[0, :N], parts1[1, :N], root1,
                                  wt2, bt2, lin2, bias2.reshape(1, H))
    parts2 = _sc_layer(g2, hb2, ea, nn2_w1, nn2_b1.reshape(1, K),
                       src, dst).reshape(NC, NA, H)

    return _final(parts2[0, :N], parts2[1, :N], root2, cls_w,
                  cls_b.reshape(1, 2))


# speedup vs baseline: 1.6564x; 1.6564x over previous
import functools

import jax
import jax.numpy as jnp
from jax import lax
from jax.experimental import pallas as pl
from jax.experimental.pallas import tpu as pltpu
from jax.experimental.pallas import tpu_sc as plsc

N = 10000
E = 160000
DN = 128
H = 16
K = 16

NC = 2
NS = 16
NW = NC * NS
EPW = E // NW
CH = 100
NCHUNK = EPW // CH
NBUF = 2
NA = 10240
RPT = NA // NS
ZB = 64
GW = K * H


def _make_sc_kernel():
    mesh = plsc.VectorSubcoreMesh(core_axis_name="c", subcore_axis_name="s")

    @functools.partial(
        pl.kernel,
        out_type=jax.ShapeDtypeStruct((NC, NS, RPT, H), jnp.float32),
        mesh=mesh,
        scratch_types=[
            pltpu.VMEM((NCHUNK, CH), jnp.int32),
            pltpu.VMEM((NCHUNK, CH), jnp.int32),
            pltpu.VMEM((K, K), jnp.float32),
            pltpu.VMEM((1, K), jnp.float32),
            pltpu.VMEM((NBUF, CH, K), jnp.float32),
            pltpu.VMEM((NBUF, CH, GW), jnp.float32),
            pltpu.VMEM((NBUF, CH, H), jnp.float32),
            pltpu.VMEM((NBUF, CH, H), jnp.float32),
            pltpu.VMEM((ZB, H), jnp.float32),
            pltpu.VMEM_SHARED((NA, H), jnp.float32),
            pltpu.SemaphoreType.DMA,
            pltpu.SemaphoreType.DMA,
            pltpu.SemaphoreType.DMA,
            pltpu.SemaphoreType.DMA,
            pltpu.SemaphoreType.DMA,
            pltpu.SemaphoreType.DMA,
        ],
        compiler_params=pltpu.CompilerParams(use_tc_tiling_on_sc=False),
    )
    def sc_kernel(g_hbm, hb_hbm, ea_hbm, w_hbm, b_hbm, src_hbm, dst_hbm,
                  parts_hbm,
                  src_v, dst_v, w_v, b_v, ea_v, g_v, hb_v, msg_v, zero_v,
                  agg_sh, gsem0, gsem1, bsem0, bsem1, esem0, esem1):
        cid = lax.axis_index("c")
        sid = lax.axis_index("s")
        wid = sid * NC + cid
        gsems = (gsem0, gsem1)
        bsems = (bsem0, bsem1)
        esems = (esem0, esem1)

        pltpu.sync_copy(src_hbm.at[wid], src_v)
        pltpu.sync_copy(dst_hbm.at[wid], dst_v)
        pltpu.sync_copy(w_hbm, w_v)
        pltpu.sync_copy(b_hbm, b_v)
        wrows = [w_v[i, :] for i in range(K)]
        bvec = b_v[0, :]

        def _zfill(j, c):
            zero_v[j, :] = jnp.zeros((H,), jnp.float32)
            return c
        lax.fori_loop(0, ZB, _zfill, 0)

        def _zcopy(j, c):
            pltpu.sync_copy(zero_v, agg_sh.at[pl.ds(sid * RPT + j * ZB, ZB)])
            return c
        lax.fori_loop(0, RPT // ZB, _zcopy, 0)
        plsc.subcore_barrier()

        def _start(t, b):
            pltpu.async_copy(g_hbm.at[src_v.at[t]], g_v.at[b], gsems[b])
            pltpu.async_copy(hb_hbm.at[src_v.at[t]], hb_v.at[b], bsems[b])
            pltpu.async_copy(ea_hbm.at[wid, t], ea_v.at[b], esems[b])

        def _wait(t, b):
            pltpu.make_async_copy(g_hbm.at[src_v.at[t]], g_v.at[b],
                                  gsems[b]).wait()
            pltpu.make_async_copy(hb_hbm.at[src_v.at[t]], hb_v.at[b],
                                  bsems[b]).wait()
            pltpu.make_async_copy(ea_hbm.at[wid, t], ea_v.at[b],
                                  esems[b]).wait()

        def _compute(b):
            def _edge(e, c):
                earow = ea_v[b, e, :]
                hacc = [bvec + earow[0] * wrows[0], earow[1] * wrows[1],
                        earow[2] * wrows[2], earow[3] * wrows[3]]
                for i in range(4, K):
                    hacc[i % 4] = hacc[i % 4] + earow[i] * wrows[i]
                hrow = jnp.maximum((hacc[0] + hacc[1]) + (hacc[2] + hacc[3]),
                                   0.0)

                def term(k):
                    return hrow[k] * g_v[b, e, pl.ds(k * H, H)]
                acc = [hb_v[b, e, :] + term(0), term(1), term(2), term(3)]
                for k in range(4, K):
                    acc[k % 4] = acc[k % 4] + term(k)
                msg_v[b, e, :] = (acc[0] + acc[1]) + (acc[2] + acc[3])
                return c
            lax.fori_loop(0, CH, _edge, 0)

        for b in range(NBUF):
            _start(b, b)

        def _group(gi, c):
            for b in range(NBUF):
                t = gi * NBUF + b
                _wait(t, b)
                _compute(b)
                pltpu.sync_copy(msg_v.at[b], agg_sh.at[dst_v.at[t]], add=True)

                @pl.when(t + NBUF < NCHUNK)
                def _():
                    _start(t + NBUF, b)
            return c
        lax.fori_loop(0, NCHUNK // NBUF, _group, 0)

        plsc.subcore_barrier()
        pltpu.sync_copy(agg_sh.at[pl.ds(sid * RPT, RPT)],
                        parts_hbm.at[cid, sid])

    return sc_kernel


_sc_layer = _make_sc_kernel()


_BN = 2000


def _node_pre_body(x_ref, wt_ref, bt_ref, lin_ref, bias_ref,
                   g_ref, hb_ref, root_ref):
    xb = x_ref[...]
    g_ref[...] = jnp.dot(xb, wt_ref[...], preferred_element_type=jnp.float32)
    hb_ref[...] = jnp.dot(xb, bt_ref[...], preferred_element_type=jnp.float32)
    root_ref[...] = (
        jnp.dot(xb, lin_ref[...], preferred_element_type=jnp.float32)
        + bias_ref[...])


def _combine_pre_body(a0_ref, a1_ref, root_ref, wt_ref, bt_ref, lin_ref,
                      bias_ref, g_ref, hb_ref, root2_ref):
    hn = jnp.maximum(a0_ref[...] + a1_ref[...] + root_ref[...], 0.0)
    g_ref[...] = jnp.dot(hn, wt_ref[...], preferred_element_type=jnp.float32)
    hb_ref[...] = jnp.dot(hn, bt_ref[...], preferred_element_type=jnp.float32)
    root2_ref[...] = (
        jnp.dot(hn, lin_ref[...], preferred_element_type=jnp.float32)
        + bias_ref[...])


def _final_body(a0_ref, a1_ref, root_ref, cw_ref, cb_ref, out_ref):
    hn = jnp.maximum(a0_ref[...] + a1_ref[...] + root_ref[...], 0.0)
    logits = (jnp.dot(hn, cw_ref[...], preferred_element_type=jnp.float32)
              + cb_ref[...])
    m = jnp.max(logits, axis=1, keepdims=True)
    z = logits - m
    lse = jnp.log(jnp.sum(jnp.exp(z), axis=1, keepdims=True))
    out_ref[...] = z - lse


def _full(shape):
    return pl.BlockSpec(shape, lambda i: (0,) * len(shape))


def _node_pre(x, wt, bt, lin, bias):
    d = x.shape[1]
    grid = (N // _BN,)
    return pl.pallas_call(
        _node_pre_body,
        grid=grid,
        in_specs=[
            pl.BlockSpec((_BN, d), lambda i: (i, 0)),
            _full((d, GW)), _full((d, H)), _full((d, H)), _full((1, H)),
        ],
        out_specs=[pl.BlockSpec((_BN, GW), lambda i: (i, 0)),
                   pl.BlockSpec((_BN, H), lambda i: (i, 0)),
                   pl.BlockSpec((_BN, H), lambda i: (i, 0))],
        out_shape=[jax.ShapeDtypeStruct((N, GW), jnp.float32),
                   jax.ShapeDtypeStruct((N, H), jnp.float32),
                   jax.ShapeDtypeStruct((N, H), jnp.float32)],
    )(x, wt, bt, lin, bias)


def _combine_pre(a0, a1, root, wt, bt, lin, bias):
    grid = (N // _BN,)
    return pl.pallas_call(
        _combine_pre_body,
        grid=grid,
        in_specs=[
            pl.BlockSpec((_BN, H), lambda i: (i, 0)),
            pl.BlockSpec((_BN, H), lambda i: (i, 0)),
            pl.BlockSpec((_BN, H), lambda i: (i, 0)),
            _full((H, GW)), _full((H, H)), _full((H, H)), _full((1, H)),
        ],
        out_specs=[pl.BlockSpec((_BN, GW), lambda i: (i, 0)),
                   pl.BlockSpec((_BN, H), lambda i: (i, 0)),
                   pl.BlockSpec((_BN, H), lambda i: (i, 0))],
        out_shape=[jax.ShapeDtypeStruct((N, GW), jnp.float32),
                   jax.ShapeDtypeStruct((N, H), jnp.float32),
                   jax.ShapeDtypeStruct((N, H), jnp.float32)],
    )(a0, a1, root, wt, bt, lin, bias)


def _final(a0, a1, root, cw, cb):
    grid = (N // _BN,)
    return pl.pallas_call(
        _final_body,
        grid=grid,
        in_specs=[
            pl.BlockSpec((_BN, H), lambda i: (i, 0)),
            pl.BlockSpec((_BN, H), lambda i: (i, 0)),
            pl.BlockSpec((_BN, H), lambda i: (i, 0)),
            _full((H, 2)), _full((1, 2)),
        ],
        out_specs=pl.BlockSpec((_BN, 2), lambda i: (i, 0)),
        out_shape=jax.ShapeDtypeStruct((N, 2), jnp.float32),
    )(a0, a1, root, cw, cb)


def kernel(x, edge_index, edge_attr, nn1_w1, nn1_b1, nn1_w2, nn1_b2, lin1,
           bias1, nn2_w1, nn2_b1, nn2_w2, nn2_b2, lin2, bias2, cls_w, cls_b):
    wt1 = nn1_w2.reshape(K, DN, H).transpose(1, 0, 2).reshape(DN, K * H)
    bt1 = nn1_b2.reshape(DN, H)
    wt2 = nn2_w2.reshape(K, H, H).transpose(1, 0, 2).reshape(H, K * H)
    bt2 = nn2_b2.reshape(H, H)

    src = edge_index[0].reshape(NW, NCHUNK, CH)
    dst = edge_index[1].reshape(NW, NCHUNK, CH)
    ea = edge_attr.reshape(NW, NCHUNK, CH, K)

    g1, hb1, root1 = _node_pre(x, wt1, bt1, lin1, bias1.reshape(1, H))
    parts1 = _sc_layer(g1, hb1, ea, nn1_w1, nn1_b1.reshape(1, K),
                       src, dst).reshape(NC, NA, H)

    g2, hb2, root2 = _combine_pre(parts1[0, :N], parts1[1, :N], root1,
                                  wt2, bt2, lin2, bias2.reshape(1, H))
    parts2 = _sc_layer(g2, hb2, ea, nn2_w1, nn2_b1.reshape(1, K),
                       src, dst).reshape(NC, NA, H)

    return _final(parts2[0, :N], parts2[1, :N], root2, cls_w,
                  cls_b.reshape(1, 2))
